# baseline (speedup 1.0000x reference)
import dataclasses
import functools
import math

import jax
import jax.numpy as jnp
from jax import lax
from jax.experimental import pallas as pl
from jax.experimental.pallas import tpu as pltpu
from jax.experimental.pallas import tpu_sc as plsc

TOP_P = 0.5
N = 2048
D = 1024
E = 8
H = 2048
DO = 1024
BN = 256
NB = N // BN
SC_CH = 32
SUB_RANGE = N // 4
NC = 2


def _gate_kernel(x_ref, gw_ref, gb_ref, c_ref, mask_ref, loss_ref, *, top_p):
    n = x_ref.shape[0]
    e = gw_ref.shape[1]
    logits = jnp.dot(x_ref[...], gw_ref[...],
                     preferred_element_type=jnp.float32) + gb_ref[...]
    lt = logits.T

    m = jnp.max(lt, axis=0, keepdims=True)
    un = jnp.exp(lt - m)
    probs = un / jnp.sum(un, axis=0, keepdims=True)

    ent = -jnp.sum(probs * jnp.log(probs + 1e-08), axis=0, keepdims=True)
    loss_ref[...] = jnp.sum(ent, axis=1, keepdims=True) / n

    sub = lax.broadcasted_iota(jnp.int32, (e, n), 0)
    work = probs
    acc = jnp.zeros((1, n), jnp.float32)
    count = jnp.zeros((1, n), jnp.int32)
    for i in range(e):
        cur = jnp.max(work, axis=0, keepdims=True)
        acc = acc + cur
        sel = acc <= top_p
        if i == 0:
            sel = jnp.ones_like(sel)
        count = count + sel.astype(jnp.int32)
        elig = work == cur
        first = jnp.min(jnp.where(elig, sub, e), axis=0, keepdims=True)
        work = jnp.where(sub == first, -jnp.inf, work)

    cols = []
    for j in range(e):
        pj = probs[j:j + 1, :]
        gt = jnp.sum((probs > pj).astype(jnp.int32), axis=0, keepdims=True)
        eq = jnp.sum(((probs == pj) & (sub < j)).astype(jnp.int32),
                     axis=0, keepdims=True)
        cols.append(gt + eq)
    rank = jnp.concatenate(cols, axis=0)

    maski = (rank < count).astype(jnp.int32)
    mask_ref[...] = maski
    maskf = maski.astype(jnp.float32)
    mp = probs * maskf
    c_ref[...] = (mp / jnp.sum(mp, axis=0, keepdims=True)).T


def _route_build(mask_hbm, tok_hbm, cnt_hbm, mask_v, list_v, cnt_v, sem):
    wid = lax.axis_index("s") * NC + lax.axis_index("c")

    @pl.when(wid < E)
    def _():
        pltpu.async_copy(mask_hbm.at[wid], mask_v, sem).wait()

        @pl.loop(0, N, step=16)
        def _(i):
            list_v[pl.ds(i, 16)] = jnp.zeros((16,), jnp.int32)

        def body(ci, run):
            mv = mask_v[pl.ds(ci * 16, 16)]
            incl = plsc.cumsum(mv)
            posv = (run + incl) - mv
            tokv = lax.broadcasted_iota(jnp.int32, (16,), 0) + ci * 16
            plsc.store_scatter(list_v, [posv], tokv, mask=mv != 0)
            return run + jnp.sum(mv)

        total = lax.fori_loop(0, N // 16, body, jnp.int32(0))
        pltpu.async_copy(list_v, tok_hbm.at[pl.ds(wid * N, N)], sem).wait()
        cnt_v[...] = jnp.full((16,), total, jnp.int32)
        pltpu.async_copy(cnt_v, cnt_hbm.at[wid], sem).wait()


def _sc_gather(x_hbm, tok_hbm, cnt_hbm, xg_hbm, idx_v, rows_v, cnt_s, sem):
    wid = lax.axis_index("s") * NC + lax.axis_index("c")
    ew = wid // 4
    q = wid % 4
    base = q * SUB_RANGE

    @pl.when(wid < E * 4)
    def _():
        pltpu.async_copy(cnt_hbm.at[ew], cnt_s, sem).wait()
        cnt = jnp.max(cnt_s[...])
        nact = jnp.clip(cnt - base, 0, SUB_RANGE)
        nact = ((nact + BN - 1) // BN) * BN
        pltpu.async_copy(tok_hbm.at[pl.ds(ew * N + base, SUB_RANGE)], idx_v,
                         sem).wait()
        for ci in range(SUB_RANGE // SC_CH):
            @pl.when(ci * SC_CH < nact)
            def _(ci=ci):
                pltpu.async_copy(
                    x_hbm.at[idx_v.at[pl.ds(ci * SC_CH, SC_CH)]], rows_v,
                    sem).wait()
                pltpu.async_copy(
                    rows_v,
                    xg_hbm.at[pl.ds(ew * N + base + ci * SC_CH, SC_CH)],
                    sem).wait()


def _group_kernel(cnt_ref, xg_ref, w1_ref, b1_ref, w2_ref, b2_ref, c_ref,
                  tok_ref, out_ref, *, inv_sqrt2):
    e = pl.program_id(0)
    i = pl.program_id(1)

    @pl.when((e == 0) & (i == 0))
    def _():
        out_ref[...] = jnp.zeros_like(out_ref)

    cnt = cnt_ref[e, 0]

    @pl.when(i * BN < cnt)
    def _():
        xb = xg_ref[...]
        h = jnp.dot(xb, w1_ref[0], preferred_element_type=jnp.float32) \
            + b1_ref[0]
        h = 0.5 * h * (1.0 + lax.erf(h * inv_sqrt2))
        o = jnp.dot(h, w2_ref[0], preferred_element_type=jnp.float32) \
            + b2_ref[0]

        tokblk = tok_ref[:, pl.ds(pl.multiple_of(i * BN, BN), BN)]
        sub = lax.broadcasted_iota(jnp.int32, (E, BN), 0)
        tokv = jnp.sum(jnp.where(sub == e, tokblk, 0), axis=0, keepdims=True)
        pcol = lax.broadcasted_iota(jnp.int32, (1, BN), 1) + i * BN
        valid = pcol < cnt

        lane = lax.broadcasted_iota(jnp.int32, (N, E), 1)
        cv = jnp.sum(jnp.where(lane == e, c_ref[...], 0.0), axis=1,
                     keepdims=True)
        tmat = lax.broadcasted_iota(jnp.int32, (N, BN), 0)
        pw = jnp.where((tmat == tokv) & valid, cv, 0.0)
        out_ref[...] += jnp.dot(pw, o, preferred_element_type=jnp.float32)


def kernel(x, gate_w, gate_b, w1, b1, w2, b2):
    c, mask, loss = pl.pallas_call(
        functools.partial(_gate_kernel, top_p=TOP_P),
        out_shape=[
            jax.ShapeDtypeStruct((N, E), jnp.float32),
            jax.ShapeDtypeStruct((E, N), jnp.int32),
            jax.ShapeDtypeStruct((1, 1), jnp.float32),
        ],
    )(x, gate_w, gate_b.reshape(1, E))

    mesh = plsc.VectorSubcoreMesh(core_axis_name="c", subcore_axis_name="s")
    sc_params = pltpu.CompilerParams()
    if "needs_layout_passes" in pltpu.CompilerParams.__dataclass_fields__:
        sc_params = dataclasses.replace(sc_params, needs_layout_passes=False)
    tok, cnt = pl.kernel(
        _route_build,
        compiler_params=sc_params,
        out_type=[
            jax.ShapeDtypeStruct((E * N,), jnp.int32),
            jax.ShapeDtypeStruct((E, 16), jnp.int32),
        ],
        mesh=mesh,
        scratch_types=[
            pltpu.VMEM((N,), jnp.int32),
            pltpu.VMEM((N,), jnp.int32),
            pltpu.VMEM((16,), jnp.int32),
            pltpu.SemaphoreType.DMA,
        ],
    )(mask)

    xg = pl.kernel(
        _sc_gather,
        compiler_params=sc_params,
        out_type=jax.ShapeDtypeStruct((E * N, D), jnp.float32),
        mesh=mesh,
        scratch_types=[
            pltpu.VMEM((SUB_RANGE,), jnp.int32),
            pltpu.VMEM((SC_CH, D), jnp.float32),
            pltpu.VMEM((16,), jnp.int32),
            pltpu.SemaphoreType.DMA,
        ],
    )(x, tok, cnt)

    out = pl.pallas_call(
        functools.partial(_group_kernel, inv_sqrt2=1.0 / math.sqrt(2.0)),
        grid_spec=pltpu.PrefetchScalarGridSpec(
            num_scalar_prefetch=1,
            grid=(E, NB),
            in_specs=[
                pl.BlockSpec(
                    (BN, D),
                    lambda e, i, cnt_ref: (
                        e * NB + jnp.minimum(
                            i,
                            jnp.maximum(
                                (cnt_ref[e, 0] + BN - 1) // BN - 1, 0)),
                        0)),
                pl.BlockSpec((1, D, H), lambda e, i, cnt_ref: (e, 0, 0)),
                pl.BlockSpec((1, 1, H), lambda e, i, cnt_ref: (e, 0, 0)),
                pl.BlockSpec((1, H, DO), lambda e, i, cnt_ref: (e, 0, 0)),
                pl.BlockSpec((1, 1, DO), lambda e, i, cnt_ref: (e, 0, 0)),
                pl.BlockSpec((N, E), lambda e, i, cnt_ref: (0, 0)),
                pl.BlockSpec((E, N), lambda e, i, cnt_ref: (0, 0)),
            ],
            out_specs=pl.BlockSpec((N, DO), lambda e, i, cnt_ref: (0, 0)),
        ),
        out_shape=jax.ShapeDtypeStruct((N, DO), jnp.float32),
    )(cnt, xg, w1, b1.reshape(E, 1, H), w2, b2.reshape(E, 1, DO), c,
      tok.reshape(E, N))

    return (out, loss.reshape(()))


# speedup vs baseline: 1.3085x; 1.3085x over previous
import dataclasses
import functools
import math

import jax
import jax.numpy as jnp
from jax import lax
from jax.experimental import pallas as pl
from jax.experimental.pallas import tpu as pltpu
from jax.experimental.pallas import tpu_sc as plsc

TOP_P = 0.5
N = 2048
D = 1024
E = 8
H = 2048
DO = 1024
BN = 256
NB = N // BN
SC_CH = 32
SUB_RANGE = N // 4
NC = 2


def _gate_kernel(x_ref, gw_ref, gb_ref, c_ref, mask_ref, loss_ref, *, top_p):
    n = x_ref.shape[0]
    e = gw_ref.shape[1]
    logits = jnp.dot(x_ref[...], gw_ref[...],
                     preferred_element_type=jnp.float32) + gb_ref[...]
    lt = logits.T

    m = jnp.max(lt, axis=0, keepdims=True)
    un = jnp.exp(lt - m)
    probs = un / jnp.sum(un, axis=0, keepdims=True)

    ent = -jnp.sum(probs * jnp.log(probs + 1e-08), axis=0, keepdims=True)
    loss_ref[...] = jnp.sum(ent, axis=1, keepdims=True) / n

    sub = lax.broadcasted_iota(jnp.int32, (e, n), 0)
    work = probs
    acc = jnp.zeros((1, n), jnp.float32)
    count = jnp.zeros((1, n), jnp.int32)
    for i in range(e):
        cur = jnp.max(work, axis=0, keepdims=True)
        acc = acc + cur
        sel = acc <= top_p
        if i == 0:
            sel = jnp.ones_like(sel)
        count = count + sel.astype(jnp.int32)
        elig = work == cur
        first = jnp.min(jnp.where(elig, sub, e), axis=0, keepdims=True)
        work = jnp.where(sub == first, -jnp.inf, work)

    cols = []
    for j in range(e):
        pj = probs[j:j + 1, :]
        gt = jnp.sum((probs > pj).astype(jnp.int32), axis=0, keepdims=True)
        eq = jnp.sum(((probs == pj) & (sub < j)).astype(jnp.int32),
                     axis=0, keepdims=True)
        cols.append(gt + eq)
    rank = jnp.concatenate(cols, axis=0)

    maski = (rank < count).astype(jnp.int32)
    mask_ref[...] = maski
    maskf = maski.astype(jnp.float32)
    mp = probs * maskf
    c_ref[...] = (mp / jnp.sum(mp, axis=0, keepdims=True)).T


def _route_build(mask_hbm, tok_hbm, cnt_hbm, mask_v, list_v, cnt_v, sem):
    wid = lax.axis_index("s") * NC + lax.axis_index("c")

    @pl.when(wid < E)
    def _():
        pltpu.async_copy(mask_hbm.at[wid], mask_v, sem).wait()

        @pl.loop(0, N, step=16)
        def _(i):
            list_v[pl.ds(i, 16)] = jnp.zeros((16,), jnp.int32)

        def body(ci, run):
            mv = mask_v[pl.ds(ci * 16, 16)]
            incl = plsc.cumsum(mv)
            posv = (run + incl) - mv
            tokv = lax.broadcasted_iota(jnp.int32, (16,), 0) + ci * 16
            plsc.store_scatter(list_v, [posv], tokv, mask=mv != 0)
            return run + jnp.sum(mv)

        total = lax.fori_loop(0, N // 16, body, jnp.int32(0))
        pltpu.async_copy(list_v, tok_hbm.at[pl.ds(wid * N, N)], sem).wait()
        cnt_v[...] = jnp.full((16,), total, jnp.int32)
        pltpu.async_copy(cnt_v, cnt_hbm.at[wid], sem).wait()


def _group_kernel(cnt_ref, x_ref, w1_ref, b1_ref, w2_ref, b2_ref, c_ref,
                  tok_ref, out_ref, *, inv_sqrt2):
    e = pl.program_id(0)
    i = pl.program_id(1)

    @pl.when((e == 0) & (i == 0))
    def _():
        out_ref[...] = jnp.zeros_like(out_ref)

    cnt = cnt_ref[e, 0]

    @pl.when(i * BN < cnt)
    def _():
        tokblk = tok_ref[:, pl.ds(pl.multiple_of(i * BN, BN), BN)]
        sub = lax.broadcasted_iota(jnp.int32, (E, BN), 0)
        tokv = jnp.sum(jnp.where(sub == e, tokblk, 0), axis=0, keepdims=True)
        pcol = lax.broadcasted_iota(jnp.int32, (1, BN), 1) + i * BN
        valid = pcol < cnt
        tmat = lax.broadcasted_iota(jnp.int32, (N, BN), 0)
        onehot = tmat == tokv

        g = jnp.where(onehot, 1.0, 0.0)
        xb = lax.dot_general(g, x_ref[...], (((0,), (0,)), ((), ())),
                             preferred_element_type=jnp.float32)

        h = jnp.dot(xb, w1_ref[0], preferred_element_type=jnp.float32) \
            + b1_ref[0]
        h = 0.5 * h * (1.0 + lax.erf(h * inv_sqrt2))
        o = jnp.dot(h, w2_ref[0], preferred_element_type=jnp.float32) \
            + b2_ref[0]

        lane = lax.broadcasted_iota(jnp.int32, (N, E), 1)
        cv = jnp.sum(jnp.where(lane == e, c_ref[...], 0.0), axis=1,
                     keepdims=True)
        pw = jnp.where(onehot & valid, cv, 0.0)
        out_ref[...] += jnp.dot(pw, o, preferred_element_type=jnp.float32)


def kernel(x, gate_w, gate_b, w1, b1, w2, b2):
    c, mask, loss = pl.pallas_call(
        functools.partial(_gate_kernel, top_p=TOP_P),
        out_shape=[
            jax.ShapeDtypeStruct((N, E), jnp.float32),
            jax.ShapeDtypeStruct((E, N), jnp.int32),
            jax.ShapeDtypeStruct((1, 1), jnp.float32),
        ],
    )(x, gate_w, gate_b.reshape(1, E))

    mesh = plsc.VectorSubcoreMesh(core_axis_name="c", subcore_axis_name="s")
    sc_params = pltpu.CompilerParams()
    if "needs_layout_passes" in pltpu.CompilerParams.__dataclass_fields__:
        sc_params = dataclasses.replace(sc_params, needs_layout_passes=False)
    tok, cnt = pl.kernel(
        _route_build,
        compiler_params=sc_params,
        out_type=[
            jax.ShapeDtypeStruct((E * N,), jnp.int32),
            jax.ShapeDtypeStruct((E, 16), jnp.int32),
        ],
        mesh=mesh,
        scratch_types=[
            pltpu.VMEM((N,), jnp.int32),
            pltpu.VMEM((N,), jnp.int32),
            pltpu.VMEM((16,), jnp.int32),
            pltpu.SemaphoreType.DMA,
        ],
    )(mask)

    out = pl.pallas_call(
        functools.partial(_group_kernel, inv_sqrt2=1.0 / math.sqrt(2.0)),
        grid_spec=pltpu.PrefetchScalarGridSpec(
            num_scalar_prefetch=1,
            grid=(E, NB),
            in_specs=[
                pl.BlockSpec((N, D), lambda e, i, cnt_ref: (0, 0)),
                pl.BlockSpec((1, D, H), lambda e, i, cnt_ref: (e, 0, 0)),
                pl.BlockSpec((1, 1, H), lambda e, i, cnt_ref: (e, 0, 0)),
                pl.BlockSpec((1, H, DO), lambda e, i, cnt_ref: (e, 0, 0)),
                pl.BlockSpec((1, 1, DO), lambda e, i, cnt_ref: (e, 0, 0)),
                pl.BlockSpec((N, E), lambda e, i, cnt_ref: (0, 0)),
                pl.BlockSpec((E, N), lambda e, i, cnt_ref: (0, 0)),
            ],
            out_specs=pl.BlockSpec((N, DO), lambda e, i, cnt_ref: (0, 0)),
        ),
        out_shape=jax.ShapeDtypeStruct((N, DO), jnp.float32),
    )(cnt, x, w1, b1.reshape(E, 1, H), w2, b2.reshape(E, 1, DO), c,
      tok.reshape(E, N))

    return (out, loss.reshape(()))


# speedup vs baseline: 1.3673x; 1.0449x over previous
import dataclasses
import functools
import math

import jax
import jax.numpy as jnp
from jax import lax
from jax.experimental import pallas as pl
from jax.experimental.pallas import tpu as pltpu
from jax.experimental.pallas import tpu_sc as plsc

TOP_P = 0.5
N = 2048
D = 1024
E = 8
H = 2048
DO = 1024
BN = 512
NB = N // BN
SC_CH = 32
SUB_RANGE = N // 4
NC = 2


def _gate_kernel(x_ref, gw_ref, gb_ref, c_ref, mask_ref, loss_ref, *, top_p):
    n = x_ref.shape[0]
    e = gw_ref.shape[1]
    logits = jnp.dot(x_ref[...], gw_ref[...],
                     preferred_element_type=jnp.float32) + gb_ref[...]
    lt = logits.T

    m = jnp.max(lt, axis=0, keepdims=True)
    un = jnp.exp(lt - m)
    probs = un / jnp.sum(un, axis=0, keepdims=True)

    ent = -jnp.sum(probs * jnp.log(probs + 1e-08), axis=0, keepdims=True)
    loss_ref[...] = jnp.sum(ent, axis=1, keepdims=True) / n

    sub = lax.broadcasted_iota(jnp.int32, (e, n), 0)
    work = probs
    acc = jnp.zeros((1, n), jnp.float32)
    count = jnp.zeros((1, n), jnp.int32)
    for i in range(e):
        cur = jnp.max(work, axis=0, keepdims=True)
        acc = acc + cur
        sel = acc <= top_p
        if i == 0:
            sel = jnp.ones_like(sel)
        count = count + sel.astype(jnp.int32)
        elig = work == cur
        first = jnp.min(jnp.where(elig, sub, e), axis=0, keepdims=True)
        work = jnp.where(sub == first, -jnp.inf, work)

    cols = []
    for j in range(e):
        pj = probs[j:j + 1, :]
        gt = jnp.sum((probs > pj).astype(jnp.int32), axis=0, keepdims=True)
        eq = jnp.sum(((probs == pj) & (sub < j)).astype(jnp.int32),
                     axis=0, keepdims=True)
        cols.append(gt + eq)
    rank = jnp.concatenate(cols, axis=0)

    maski = (rank < count).astype(jnp.int32)
    mask_ref[...] = maski
    maskf = maski.astype(jnp.float32)
    mp = probs * maskf
    c_ref[...] = (mp / jnp.sum(mp, axis=0, keepdims=True)).T


def _route_build(mask_hbm, tok_hbm, cnt_hbm, mask_v, list_v, cnt_v, sem):
    wid = lax.axis_index("s") * NC + lax.axis_index("c")

    @pl.when(wid < E)
    def _():
        pltpu.async_copy(mask_hbm.at[wid], mask_v, sem).wait()

        @pl.loop(0, N, step=16)
        def _(i):
            list_v[pl.ds(i, 16)] = jnp.zeros((16,), jnp.int32)

        def body(ci, run):
            mv = mask_v[pl.ds(ci * 16, 16)]
            incl = plsc.cumsum(mv)
            posv = (run + incl) - mv
            tokv = lax.broadcasted_iota(jnp.int32, (16,), 0) + ci * 16
            plsc.store_scatter(list_v, [posv], tokv, mask=mv != 0)
            return run + jnp.sum(mv)

        total = lax.fori_loop(0, N // 16, body, jnp.int32(0))
        pltpu.async_copy(list_v, tok_hbm.at[pl.ds(wid * N, N)], sem).wait()
        cnt_v[...] = jnp.full((16,), total, jnp.int32)
        pltpu.async_copy(cnt_v, cnt_hbm.at[wid], sem).wait()


def _group_kernel(cnt_ref, x_ref, w1_ref, b1_ref, w2_ref, b2_ref, c_ref,
                  tok_ref, out_ref, *, inv_sqrt2):
    e = pl.program_id(0)
    i = pl.program_id(1)

    @pl.when((e == 0) & (i == 0))
    def _():
        out_ref[...] = jnp.zeros_like(out_ref)

    cnt = cnt_ref[e, 0]

    @pl.when(i * BN < cnt)
    def _():
        tokblk = tok_ref[:, pl.ds(pl.multiple_of(i * BN, BN), BN)]
        sub = lax.broadcasted_iota(jnp.int32, (E, BN), 0)
        tokv = jnp.sum(jnp.where(sub == e, tokblk, 0), axis=0, keepdims=True)
        pcol = lax.broadcasted_iota(jnp.int32, (1, BN), 1) + i * BN
        valid = pcol < cnt
        tmat = lax.broadcasted_iota(jnp.int32, (N, BN), 0)
        onehot = tmat == tokv

        g = jnp.where(onehot, 1.0, 0.0)
        xb = lax.dot_general(g, x_ref[...], (((0,), (0,)), ((), ())),
                             preferred_element_type=jnp.float32)

        h = jnp.dot(xb, w1_ref[0], preferred_element_type=jnp.float32) \
            + b1_ref[0]
        h = 0.5 * h * (1.0 + lax.erf(h * inv_sqrt2))
        o = jnp.dot(h, w2_ref[0], preferred_element_type=jnp.float32) \
            + b2_ref[0]

        lane = lax.broadcasted_iota(jnp.int32, (N, E), 1)
        cv = jnp.sum(jnp.where(lane == e, c_ref[...], 0.0), axis=1,
                     keepdims=True)
        pw = jnp.where(onehot & valid, cv, 0.0)
        out_ref[...] += jnp.dot(pw, o, preferred_element_type=jnp.float32)


def kernel(x, gate_w, gate_b, w1, b1, w2, b2):
    c, mask, loss = pl.pallas_call(
        functools.partial(_gate_kernel, top_p=TOP_P),
        out_shape=[
            jax.ShapeDtypeStruct((N, E), jnp.float32),
            jax.ShapeDtypeStruct((E, N), jnp.int32),
            jax.ShapeDtypeStruct((1, 1), jnp.float32),
        ],
    )(x, gate_w, gate_b.reshape(1, E))

    mesh = plsc.VectorSubcoreMesh(core_axis_name="c", subcore_axis_name="s")
    sc_params = pltpu.CompilerParams()
    if "needs_layout_passes" in pltpu.CompilerParams.__dataclass_fields__:
        sc_params = dataclasses.replace(sc_params, needs_layout_passes=False)
    tok, cnt = pl.kernel(
        _route_build,
        compiler_params=sc_params,
        out_type=[
            jax.ShapeDtypeStruct((E * N,), jnp.int32),
            jax.ShapeDtypeStruct((E, 16), jnp.int32),
        ],
        mesh=mesh,
        scratch_types=[
            pltpu.VMEM((N,), jnp.int32),
            pltpu.VMEM((N,), jnp.int32),
            pltpu.VMEM((16,), jnp.int32),
            pltpu.SemaphoreType.DMA,
        ],
    )(mask)

    out = pl.pallas_call(
        functools.partial(_group_kernel, inv_sqrt2=1.0 / math.sqrt(2.0)),
        grid_spec=pltpu.PrefetchScalarGridSpec(
            num_scalar_prefetch=1,
            grid=(E, NB),
            in_specs=[
                pl.BlockSpec((N, D), lambda e, i, cnt_ref: (0, 0)),
                pl.BlockSpec((1, D, H), lambda e, i, cnt_ref: (e, 0, 0)),
                pl.BlockSpec((1, 1, H), lambda e, i, cnt_ref: (e, 0, 0)),
                pl.BlockSpec((1, H, DO), lambda e, i, cnt_ref: (e, 0, 0)),
                pl.BlockSpec((1, 1, DO), lambda e, i, cnt_ref: (e, 0, 0)),
                pl.BlockSpec((N, E), lambda e, i, cnt_ref: (0, 0)),
                pl.BlockSpec((E, N), lambda e, i, cnt_ref: (0, 0)),
            ],
            out_specs=pl.BlockSpec((N, DO), lambda e, i, cnt_ref: (0, 0)),
        ),
        out_shape=jax.ShapeDtypeStruct((N, DO), jnp.float32),
    )(cnt, x, w1, b1.reshape(E, 1, H), w2, b2.reshape(E, 1, DO), c,
      tok.reshape(E, N))

    return (out, loss.reshape(()))
